# baseline (speedup 1.0000x reference)
import functools

import jax
import jax.numpy as jnp
from jax import lax
from jax.experimental import pallas as pl
from jax.experimental.pallas import tpu as pltpu
from jax.experimental.pallas import tpu_sc as plsc

N = 10000
C = 128
K = 27
EPS = 1e-5
NC, NS = 2, 16
NW = NC * NS
CHUNK = 128
ZROWS = 64
LANES = 16
ACC_ROWS = 10240
ROWS_PER_SUB = ACC_ROWS // NS
OUT_PER_SUB = 624


def _ztransform(x, W):
    def body(x_ref, w_ref, z_ref):
        z_ref[...] = jnp.dot(x_ref[...], w_ref[0],
                             preferred_element_type=jnp.float32)

    return pl.pallas_call(
        body,
        grid=(K,),
        in_specs=[
            pl.BlockSpec((N, C), lambda k: (0, 0)),
            pl.BlockSpec((1, C, C), lambda k: (k, 0, 0)),
        ],
        out_specs=pl.BlockSpec((N, C), lambda k: (k, 0)),
        out_shape=jax.ShapeDtypeStruct((K * N, C), jnp.float32),
    )(x, W)


def _edge_scatter(z, gidx_r, dst_r, nch):
    nchp = gidx_r.shape[1]
    assert nch % 2 == 0 and nch < nchp
    mesh = plsc.VectorSubcoreMesh(core_axis_name="c", subcore_axis_name="s")

    @functools.partial(
        pl.kernel,
        out_type=jax.ShapeDtypeStruct((NC, N, C), jnp.float32),
        mesh=mesh,
        scratch_types=[
            pltpu.VMEM_SHARED((ACC_ROWS, C), jnp.float32),
            pltpu.VMEM((nchp, CHUNK), jnp.int32),
            pltpu.VMEM((2, CHUNK), jnp.int32),
            pltpu.VMEM((2, CHUNK, C), jnp.float32),
            pltpu.SemaphoreType.DMA,
            pltpu.SemaphoreType.DMA,
            pltpu.SemaphoreType.DMA,
            pltpu.SemaphoreType.DMA,
            pltpu.SemaphoreType.DMA,
            pltpu.SemaphoreType.DMA,
        ],
    )
    def body(z_hbm, gidx_hbm, dst_hbm, out_hbm, acc, gix, dixb, rows,
             sg0, sg1, sd0, sd1, ss0, ss1):
        cid = lax.axis_index("c")
        sid = lax.axis_index("s")
        wid = sid * NC + cid
        semg = (sg0, sg1)
        semd = (sd0, sd1)
        sems = (ss0, ss1)

        def zrow(r, carry):
            for j in range(C // LANES):
                rows[0, r, pl.ds(j * LANES, LANES)] = jnp.zeros((LANES,),
                                                                jnp.float32)
            return carry

        lax.fori_loop(0, CHUNK, zrow, 0)
        for t in range(ROWS_PER_SUB // CHUNK):
            pltpu.sync_copy(
                rows.at[0],
                acc.at[pl.ds(sid * ROWS_PER_SUB + t * CHUNK, CHUNK)])
        base = jnp.full((LANES,), N + sid * 8, jnp.int32)
        for j in range(CHUNK // LANES):
            ids = base + lax.rem(lax.iota(jnp.int32, LANES) + j * LANES,
                                 jnp.full((LANES,), 8, jnp.int32))
            dixb[1, pl.ds(j * LANES, LANES)] = ids
        pltpu.sync_copy(gidx_hbm.at[wid], gix)
        plsc.subcore_barrier()

        pltpu.async_copy(dst_hbm.at[wid, 0], dixb.at[0], semd[0])
        pltpu.async_copy(z_hbm.at[gix.at[0]], rows.at[0], semg[0])
        pltpu.async_copy(rows.at[1], acc.at[dixb.at[1]], sems[1], add=True)

        def outer(j, carry):
            for b in (0, 1):
                i = 2 * j + b
                nb = 1 - b
                pltpu.make_async_copy(
                    z_hbm.at[gix.at[i]], rows.at[b], semg[b]).wait()
                pltpu.make_async_copy(
                    dst_hbm.at[wid, i], dixb.at[b], semd[b]).wait()
                pltpu.async_copy(rows.at[b], acc.at[dixb.at[b]], sems[b],
                                 add=True)
                pltpu.make_async_copy(
                    rows.at[nb], acc.at[dixb.at[nb]], sems[nb]).wait()
                pltpu.async_copy(dst_hbm.at[wid, i + 1], dixb.at[nb],
                                 semd[nb])
                pltpu.async_copy(z_hbm.at[gix.at[i + 1]], rows.at[nb],
                                 semg[nb])
            return carry

        lax.fori_loop(0, nch // 2, outer, 0)
        pltpu.make_async_copy(rows.at[1], acc.at[dixb.at[1]], sems[1]).wait()
        pltpu.make_async_copy(dst_hbm.at[wid, nch], dixb.at[0],
                              semd[0]).wait()
        pltpu.make_async_copy(z_hbm.at[gix.at[nch]], rows.at[0],
                              semg[0]).wait()
        plsc.subcore_barrier()
        pltpu.sync_copy(
            acc.at[pl.ds(sid * OUT_PER_SUB, OUT_PER_SUB)],
            out_hbm.at[cid, pl.ds(sid * OUT_PER_SUB, OUT_PER_SUB)],
        )
        tail = NS * OUT_PER_SUB

        @pl.when(sid == NS - 1)
        def _():
            pltpu.sync_copy(
                acc.at[pl.ds(tail, N - tail)],
                out_hbm.at[cid, pl.ds(tail, N - tail)],
            )

    return body(z, gidx_r, dst_r)


def _bn_ztransform(partials, gamma2d, beta2d, W):
    def body(p_ref, g_ref, b_ref, w_ref, z_ref, h_ref):
        @pl.when(pl.program_id(0) == 0)
        def _():
            h = p_ref[0] + p_ref[1]
            mu = jnp.mean(h, axis=0, keepdims=True)
            var = jnp.mean((h - mu) ** 2, axis=0, keepdims=True)
            y = (h - mu) * lax.rsqrt(var + EPS) * g_ref[...] + b_ref[...]
            h_ref[...] = jnp.maximum(y, 0.0)

        z_ref[...] = jnp.dot(h_ref[...], w_ref[0],
                             preferred_element_type=jnp.float32)

    return pl.pallas_call(
        body,
        grid=(K,),
        in_specs=[
            pl.BlockSpec((2, N, C), lambda k: (0, 0, 0)),
            pl.BlockSpec((1, C), lambda k: (0, 0)),
            pl.BlockSpec((1, C), lambda k: (0, 0)),
            pl.BlockSpec((1, C, C), lambda k: (k, 0, 0)),
        ],
        out_specs=pl.BlockSpec((N, C), lambda k: (k, 0)),
        out_shape=jax.ShapeDtypeStruct((K * N, C), jnp.float32),
        scratch_shapes=[pltpu.VMEM((N, C), jnp.float32)],
    )(partials, gamma2d, beta2d, W)


def _bn_stage(partials, gamma2d, beta2d, x=None):
    residual = x is not None

    def body(p_ref, g_ref, b_ref, *rest):
        if residual:
            x_ref, o_ref = rest
        else:
            (o_ref,) = rest
        h = p_ref[0] + p_ref[1]
        mu = jnp.mean(h, axis=0, keepdims=True)
        var = jnp.mean((h - mu) ** 2, axis=0, keepdims=True)
        y = (h - mu) * lax.rsqrt(var + EPS) * g_ref[...] + b_ref[...]
        if residual:
            y = y + x_ref[...]
        o_ref[...] = jnp.maximum(y, 0.0)

    args = (partials, gamma2d, beta2d) + ((x,) if residual else ())
    return pl.pallas_call(
        body,
        out_shape=jax.ShapeDtypeStruct((N, C), jnp.float32),
    )(*args)


def kernel(x, edge_index, kernel_id, W1, gamma1, beta1, W2, gamma2, beta2):
    src = edge_index[0].astype(jnp.int32)
    dst = edge_index[1].astype(jnp.int32)
    kid = kernel_id.astype(jnp.int32)

    e = src.shape[0]
    group = NW * CHUNK
    nch = (e + group - 1) // group
    nch += nch % 2
    nchp = ((nch + 1 + 7) // 8) * 8
    epad = nchp * group
    pad = epad - e
    gidx = kid * N + src
    pad_ar = jnp.arange(pad, dtype=jnp.int32)
    gidx_p = jnp.concatenate([gidx, pad_ar % (K * N)])
    dst_p = jnp.concatenate([dst, N + pad_ar % (ACC_ROWS - N)])
    gidx_r = gidx_p.reshape(nchp, NW, CHUNK).transpose(1, 0, 2)
    dst_r = dst_p.reshape(nchp, NW, CHUNK).transpose(1, 0, 2)

    g1 = gamma1.reshape(1, C)
    b1 = beta1.reshape(1, C)
    g2 = gamma2.reshape(1, C)
    b2 = beta2.reshape(1, C)

    z1 = _ztransform(x.astype(jnp.bfloat16), W1.astype(jnp.bfloat16))
    p1 = _edge_scatter(z1, gidx_r, dst_r, nch)
    z2 = _bn_ztransform(p1, g1, b1, W2)
    p2 = _edge_scatter(z2, gidx_r, dst_r, nch)
    return _bn_stage(p2, g2, b2, x=x)


# speedup vs baseline: 1.0301x; 1.0301x over previous
import functools

import jax
import jax.numpy as jnp
from jax import lax
from jax.experimental import pallas as pl
from jax.experimental.pallas import tpu as pltpu
from jax.experimental.pallas import tpu_sc as plsc

N = 10000
C = 128
K = 27
EPS = 1e-5
NC, NS = 2, 16
NW = NC * NS
CHUNK = 128
ZROWS = 64
LANES = 16
ACC_ROWS = 10240
ROWS_PER_SUB = ACC_ROWS // NS
OUT_PER_SUB = 624


def _ztransform(x, W):
    def body(x_ref, w_ref, z_ref):
        z_ref[...] = jnp.dot(x_ref[...], w_ref[0],
                             preferred_element_type=jnp.float32)

    return pl.pallas_call(
        body,
        grid=(K,),
        in_specs=[
            pl.BlockSpec((N, C), lambda k: (0, 0)),
            pl.BlockSpec((1, C, C), lambda k: (k, 0, 0)),
        ],
        out_specs=pl.BlockSpec((N, C), lambda k: (k, 0)),
        out_shape=jax.ShapeDtypeStruct((K * N, C), jnp.float32),
    )(x, W)


def _edge_scatter(z, gidx_r, dst_r, nch):
    nchp = gidx_r.shape[1]
    assert nch % 2 == 0 and nch < nchp
    mesh = plsc.VectorSubcoreMesh(core_axis_name="c", subcore_axis_name="s")

    @functools.partial(
        pl.kernel,
        out_type=jax.ShapeDtypeStruct((NC, N, C), jnp.float32),
        mesh=mesh,
        scratch_types=[
            pltpu.VMEM_SHARED((ACC_ROWS, C), jnp.float32),
            pltpu.VMEM((nchp, CHUNK), jnp.int32),
            pltpu.VMEM((2, CHUNK), jnp.int32),
            pltpu.VMEM((2, CHUNK, C), jnp.float32),
            pltpu.SemaphoreType.DMA,
            pltpu.SemaphoreType.DMA,
            pltpu.SemaphoreType.DMA,
            pltpu.SemaphoreType.DMA,
            pltpu.SemaphoreType.DMA,
            pltpu.SemaphoreType.DMA,
        ],
    )
    def body(z_hbm, gidx_hbm, dst_hbm, out_hbm, acc, gix, dixb, rows,
             sg0, sg1, sd0, sd1, ss0, ss1):
        cid = lax.axis_index("c")
        sid = lax.axis_index("s")
        wid = sid * NC + cid
        semg = (sg0, sg1)
        semd = (sd0, sd1)
        sems = (ss0, ss1)

        def zrow(r, carry):
            for j in range(C // LANES):
                rows[0, r, pl.ds(j * LANES, LANES)] = jnp.zeros((LANES,),
                                                                jnp.float32)
            return carry

        lax.fori_loop(0, CHUNK, zrow, 0)
        for t in range(ROWS_PER_SUB // CHUNK):
            pltpu.async_copy(
                rows.at[0],
                acc.at[pl.ds(sid * ROWS_PER_SUB + t * CHUNK, CHUNK)],
                sems[0])
        base = jnp.full((LANES,), N + sid * 8, jnp.int32)
        for j in range(CHUNK // LANES):
            ids = base + lax.rem(lax.iota(jnp.int32, LANES) + j * LANES,
                                 jnp.full((LANES,), 8, jnp.int32))
            dixb[1, pl.ds(j * LANES, LANES)] = ids
        pltpu.async_copy(gidx_hbm.at[wid], gix, semd[0])
        for t in range(ROWS_PER_SUB // CHUNK):
            pltpu.make_async_copy(
                rows.at[0],
                acc.at[pl.ds(sid * ROWS_PER_SUB + t * CHUNK, CHUNK)],
                sems[0]).wait()
        pltpu.make_async_copy(gidx_hbm.at[wid], gix, semd[0]).wait()
        plsc.subcore_barrier()

        pltpu.async_copy(dst_hbm.at[wid, 0], dixb.at[0], semd[0])
        pltpu.async_copy(z_hbm.at[gix.at[0]], rows.at[0], semg[0])
        pltpu.async_copy(rows.at[1], acc.at[dixb.at[1]], sems[1], add=True)

        def outer(j, carry):
            for b in (0, 1):
                i = 2 * j + b
                nb = 1 - b
                pltpu.make_async_copy(
                    z_hbm.at[gix.at[i]], rows.at[b], semg[b]).wait()
                pltpu.make_async_copy(
                    dst_hbm.at[wid, i], dixb.at[b], semd[b]).wait()
                pltpu.async_copy(rows.at[b], acc.at[dixb.at[b]], sems[b],
                                 add=True)
                pltpu.make_async_copy(
                    rows.at[nb], acc.at[dixb.at[nb]], sems[nb]).wait()
                pltpu.async_copy(dst_hbm.at[wid, i + 1], dixb.at[nb],
                                 semd[nb])
                pltpu.async_copy(z_hbm.at[gix.at[i + 1]], rows.at[nb],
                                 semg[nb])
            return carry

        lax.fori_loop(0, nch // 2, outer, 0)
        pltpu.make_async_copy(rows.at[1], acc.at[dixb.at[1]], sems[1]).wait()
        pltpu.make_async_copy(dst_hbm.at[wid, nch], dixb.at[0],
                              semd[0]).wait()
        pltpu.make_async_copy(z_hbm.at[gix.at[nch]], rows.at[0],
                              semg[0]).wait()
        plsc.subcore_barrier()
        pltpu.sync_copy(
            acc.at[pl.ds(sid * OUT_PER_SUB, OUT_PER_SUB)],
            out_hbm.at[cid, pl.ds(sid * OUT_PER_SUB, OUT_PER_SUB)],
        )
        tail = NS * OUT_PER_SUB

        @pl.when(sid == NS - 1)
        def _():
            pltpu.sync_copy(
                acc.at[pl.ds(tail, N - tail)],
                out_hbm.at[cid, pl.ds(tail, N - tail)],
            )

    return body(z, gidx_r, dst_r)


def _bn_ztransform(partials, gamma2d, beta2d, W):
    def body(p_ref, g_ref, b_ref, w_ref, z_ref, h_ref):
        @pl.when(pl.program_id(0) == 0)
        def _():
            h = p_ref[0] + p_ref[1]
            mu = jnp.mean(h, axis=0, keepdims=True)
            var = jnp.mean((h - mu) ** 2, axis=0, keepdims=True)
            y = (h - mu) * lax.rsqrt(var + EPS) * g_ref[...] + b_ref[...]
            h_ref[...] = jnp.maximum(y, 0.0)

        z_ref[...] = jnp.dot(h_ref[...], w_ref[0],
                             preferred_element_type=jnp.float32)

    return pl.pallas_call(
        body,
        grid=(K,),
        in_specs=[
            pl.BlockSpec((2, N, C), lambda k: (0, 0, 0)),
            pl.BlockSpec((1, C), lambda k: (0, 0)),
            pl.BlockSpec((1, C), lambda k: (0, 0)),
            pl.BlockSpec((1, C, C), lambda k: (k, 0, 0)),
        ],
        out_specs=pl.BlockSpec((N, C), lambda k: (k, 0)),
        out_shape=jax.ShapeDtypeStruct((K * N, C), jnp.float32),
        scratch_shapes=[pltpu.VMEM((N, C), jnp.float32)],
    )(partials, gamma2d, beta2d, W)


def _bn_stage(partials, gamma2d, beta2d, x=None):
    residual = x is not None

    def body(p_ref, g_ref, b_ref, *rest):
        if residual:
            x_ref, o_ref = rest
        else:
            (o_ref,) = rest
        h = p_ref[0] + p_ref[1]
        mu = jnp.mean(h, axis=0, keepdims=True)
        var = jnp.mean((h - mu) ** 2, axis=0, keepdims=True)
        y = (h - mu) * lax.rsqrt(var + EPS) * g_ref[...] + b_ref[...]
        if residual:
            y = y + x_ref[...]
        o_ref[...] = jnp.maximum(y, 0.0)

    args = (partials, gamma2d, beta2d) + ((x,) if residual else ())
    return pl.pallas_call(
        body,
        out_shape=jax.ShapeDtypeStruct((N, C), jnp.float32),
    )(*args)


def kernel(x, edge_index, kernel_id, W1, gamma1, beta1, W2, gamma2, beta2):
    src = edge_index[0].astype(jnp.int32)
    dst = edge_index[1].astype(jnp.int32)
    kid = kernel_id.astype(jnp.int32)

    e = src.shape[0]
    group = NW * CHUNK
    nch = (e + group - 1) // group
    nch += nch % 2
    nchp = ((nch + 1 + 7) // 8) * 8
    epad = nchp * group
    pad = epad - e
    gidx = kid * N + src
    pad_ar = jnp.arange(pad, dtype=jnp.int32)
    gidx_p = jnp.concatenate([gidx, pad_ar % (K * N)])
    dst_p = jnp.concatenate([dst, N + pad_ar % (ACC_ROWS - N)])
    gidx_r = gidx_p.reshape(nchp, NW, CHUNK).transpose(1, 0, 2)
    dst_r = dst_p.reshape(nchp, NW, CHUNK).transpose(1, 0, 2)

    g1 = gamma1.reshape(1, C)
    b1 = beta1.reshape(1, C)
    g2 = gamma2.reshape(1, C)
    b2 = beta2.reshape(1, C)

    z1 = _ztransform(x, W1)
    p1 = _edge_scatter(z1, gidx_r, dst_r, nch)
    z2 = _bn_ztransform(p1, g1, b1, W2)
    p2 = _edge_scatter(z2, gidx_r, dst_r, nch)
    return _bn_stage(p2, g2, b2, x=x)


# speedup vs baseline: 1.0318x; 1.0016x over previous
import functools

import jax
import jax.numpy as jnp
from jax import lax
from jax.experimental import pallas as pl
from jax.experimental.pallas import tpu as pltpu
from jax.experimental.pallas import tpu_sc as plsc

N = 10000
C = 128
K = 27
EPS = 1e-5
NC, NS = 2, 16
NW = NC * NS
CHUNK = 128
LANES = 16
ACC_ROWS = 10240
ROWS_PER_SUB = ACC_ROWS // NS
OUT_PER_SUB = 624


def _ztransform(x, W):
    def body(x_ref, w_ref, z_ref):
        z_ref[...] = jnp.dot(x_ref[...], w_ref[0],
                             preferred_element_type=jnp.float32)

    return pl.pallas_call(
        body,
        grid=(K,),
        in_specs=[
            pl.BlockSpec((N, C), lambda k: (0, 0)),
            pl.BlockSpec((1, C, C), lambda k: (k, 0, 0)),
        ],
        out_specs=pl.BlockSpec((N, C), lambda k: (k, 0)),
        out_shape=jax.ShapeDtypeStruct((K * N, C), jnp.float32),
    )(x, W)


def _edge_scatter(z, gidx_r, dst_r, nch):
    nchp = gidx_r.shape[1]
    assert nch % 2 == 0 and nch < nchp
    mesh = plsc.VectorSubcoreMesh(core_axis_name="c", subcore_axis_name="s")

    @functools.partial(
        pl.kernel,
        out_type=jax.ShapeDtypeStruct((NC, N, C), jnp.float32),
        mesh=mesh,
        scratch_types=[
            pltpu.VMEM_SHARED((ACC_ROWS, C), jnp.float32),
            pltpu.VMEM((nchp, CHUNK), jnp.int32),
            pltpu.VMEM((2, CHUNK), jnp.int32),
            pltpu.VMEM((2, CHUNK, C), jnp.float32),
            pltpu.SemaphoreType.DMA,
            pltpu.SemaphoreType.DMA,
            pltpu.SemaphoreType.DMA,
            pltpu.SemaphoreType.DMA,
            pltpu.SemaphoreType.DMA,
            pltpu.SemaphoreType.DMA,
        ],
    )
    def body(z_hbm, gidx_hbm, dst_hbm, out_hbm, acc, gix, dixb, rows,
             sg0, sg1, sd0, sd1, ss0, ss1):
        cid = lax.axis_index("c")
        sid = lax.axis_index("s")
        wid = sid * NC + cid
        semg = (sg0, sg1)
        semd = (sd0, sd1)
        sems = (ss0, ss1)

        def zrow(r, carry):
            for j in range(C // LANES):
                rows[0, r, pl.ds(j * LANES, LANES)] = jnp.zeros((LANES,),
                                                                jnp.float32)
            return carry

        lax.fori_loop(0, CHUNK, zrow, 0)
        for t in range(ROWS_PER_SUB // CHUNK):
            pltpu.async_copy(
                rows.at[0],
                acc.at[pl.ds(sid * ROWS_PER_SUB + t * CHUNK, CHUNK)],
                sems[0])
        base = jnp.full((LANES,), N + sid * 8, jnp.int32)
        for j in range(CHUNK // LANES):
            ids = base + lax.rem(lax.iota(jnp.int32, LANES) + j * LANES,
                                 jnp.full((LANES,), 8, jnp.int32))
            dixb[1, pl.ds(j * LANES, LANES)] = ids
        pltpu.async_copy(gidx_hbm.at[wid], gix, semd[0])
        for t in range(ROWS_PER_SUB // CHUNK):
            pltpu.make_async_copy(
                rows.at[0],
                acc.at[pl.ds(sid * ROWS_PER_SUB + t * CHUNK, CHUNK)],
                sems[0]).wait()
        pltpu.make_async_copy(gidx_hbm.at[wid], gix, semd[0]).wait()
        plsc.subcore_barrier()

        pltpu.async_copy(dst_hbm.at[wid, 0], dixb.at[0], semd[0])
        pltpu.async_copy(z_hbm.at[gix.at[0]], rows.at[0], semg[0])
        pltpu.async_copy(rows.at[1], acc.at[dixb.at[1]], sems[1], add=True)

        def outer(j, carry):
            for b in (0, 1):
                i = 2 * j + b
                nb = 1 - b
                pltpu.make_async_copy(
                    z_hbm.at[gix.at[i]], rows.at[b], semg[b]).wait()
                pltpu.make_async_copy(
                    dst_hbm.at[wid, i], dixb.at[b], semd[b]).wait()
                pltpu.async_copy(rows.at[b], acc.at[dixb.at[b]], sems[b],
                                 add=True)
                pltpu.make_async_copy(
                    rows.at[nb], acc.at[dixb.at[nb]], sems[nb]).wait()
                pltpu.async_copy(dst_hbm.at[wid, i + 1], dixb.at[nb],
                                 semd[nb])
                pltpu.async_copy(z_hbm.at[gix.at[i + 1]], rows.at[nb],
                                 semg[nb])
            return carry

        lax.fori_loop(0, nch // 2, outer, 0)
        pltpu.make_async_copy(rows.at[1], acc.at[dixb.at[1]], sems[1]).wait()
        pltpu.make_async_copy(dst_hbm.at[wid, nch], dixb.at[0],
                              semd[0]).wait()
        pltpu.make_async_copy(z_hbm.at[gix.at[nch]], rows.at[0],
                              semg[0]).wait()
        plsc.subcore_barrier()
        pltpu.sync_copy(
            acc.at[pl.ds(sid * OUT_PER_SUB, OUT_PER_SUB)],
            out_hbm.at[cid, pl.ds(sid * OUT_PER_SUB, OUT_PER_SUB)],
        )
        tail = NS * OUT_PER_SUB

        @pl.when(sid == NS - 1)
        def _():
            pltpu.sync_copy(
                acc.at[pl.ds(tail, N - tail)],
                out_hbm.at[cid, pl.ds(tail, N - tail)],
            )

    return body(z, gidx_r, dst_r)


def _bn_ztransform(partials, gamma2d, beta2d, W):
    def body(p_ref, g_ref, b_ref, w_ref, z_ref, h_ref):
        @pl.when(pl.program_id(0) == 0)
        def _():
            h = p_ref[0] + p_ref[1]
            mu = jnp.mean(h, axis=0, keepdims=True)
            var = jnp.mean((h - mu) ** 2, axis=0, keepdims=True)
            y = (h - mu) * lax.rsqrt(var + EPS) * g_ref[...] + b_ref[...]
            h_ref[...] = jnp.maximum(y, 0.0)

        z_ref[...] = jnp.dot(h_ref[...], w_ref[0],
                             preferred_element_type=jnp.float32)

    return pl.pallas_call(
        body,
        grid=(K,),
        in_specs=[
            pl.BlockSpec((2, N, C), lambda k: (0, 0, 0)),
            pl.BlockSpec((1, C), lambda k: (0, 0)),
            pl.BlockSpec((1, C), lambda k: (0, 0)),
            pl.BlockSpec((1, C, C), lambda k: (k, 0, 0)),
        ],
        out_specs=pl.BlockSpec((N, C), lambda k: (k, 0)),
        out_shape=jax.ShapeDtypeStruct((K * N, C), jnp.float32),
        scratch_shapes=[pltpu.VMEM((N, C), jnp.float32)],
    )(partials, gamma2d, beta2d, W)


def _bn_stage(partials, gamma2d, beta2d, x=None):
    residual = x is not None

    def body(p_ref, g_ref, b_ref, *rest):
        if residual:
            x_ref, o_ref = rest
        else:
            (o_ref,) = rest
        h = p_ref[0] + p_ref[1]
        mu = jnp.mean(h, axis=0, keepdims=True)
        var = jnp.mean((h - mu) ** 2, axis=0, keepdims=True)
        y = (h - mu) * lax.rsqrt(var + EPS) * g_ref[...] + b_ref[...]
        if residual:
            y = y + x_ref[...]
        o_ref[...] = jnp.maximum(y, 0.0)

    args = (partials, gamma2d, beta2d) + ((x,) if residual else ())
    return pl.pallas_call(
        body,
        out_shape=jax.ShapeDtypeStruct((N, C), jnp.float32),
    )(*args)


def kernel(x, edge_index, kernel_id, W1, gamma1, beta1, W2, gamma2, beta2):
    src = edge_index[0].astype(jnp.int32)
    dst = edge_index[1].astype(jnp.int32)
    kid = kernel_id.astype(jnp.int32)

    e = src.shape[0]
    group = NW * CHUNK
    nch = (e + group - 1) // group
    nch += nch % 2
    nchp = ((nch + 1 + 7) // 8) * 8
    epad = nchp * group
    pad = epad - e
    gidx = kid * N + src
    pad_ar = jnp.arange(pad, dtype=jnp.int32)
    gidx_p = jnp.concatenate([gidx, pad_ar % (K * N)])
    dst_p = jnp.concatenate([dst, N + pad_ar % (ACC_ROWS - N)])
    gidx_r = gidx_p.reshape(nchp, NW, CHUNK).transpose(1, 0, 2)
    dst_r = dst_p.reshape(nchp, NW, CHUNK).transpose(1, 0, 2)

    g1 = gamma1.reshape(1, C)
    b1 = beta1.reshape(1, C)
    g2 = gamma2.reshape(1, C)
    b2 = beta2.reshape(1, C)

    z1 = _ztransform(x, W1)
    p1 = _edge_scatter(z1, gidx_r, dst_r, nch)
    z2 = _bn_ztransform(p1, g1, b1, W2)
    p2 = _edge_scatter(z2, gidx_r, dst_r, nch)
    return _bn_stage(p2, g2, b2, x=x)
